# baseline (speedup 1.0000x reference)
import functools

import jax
import jax.numpy as jnp
from jax.experimental import pallas as pl
from jax.experimental.pallas import tpu as pltpu

_MASK_W = 512


def _ngram_kernel(tokens_ref, lprobs_hbm, out_hbm, blk, sem_big, sem_in, sem_out):
    big = pltpu.make_async_copy(
        lprobs_hbm.at[:, _MASK_W:], out_hbm.at[:, _MASK_W:], sem_big)
    big.start()
    cin = pltpu.make_async_copy(lprobs_hbm.at[:, :_MASK_W], blk, sem_in)
    cin.start()

    T = tokens_ref[...]
    L = T.shape[1]
    t0 = T[:, L - 3:L - 2]
    t1 = T[:, L - 2:L - 1]
    b = jnp.roll(T, -1, axis=1)
    c = jnp.roll(T, -2, axis=1)
    pos = jax.lax.broadcasted_iota(jnp.int32, T.shape, 1)
    match = (pos < (L - 3)) & (T == t0) & (b == t1)
    pw = jnp.int32(1) << (c & 31)
    lo = jnp.where(match & (c < 32), pw, 0)
    hi = jnp.where(match & (c >= 32), pw, 0)
    w = L
    while w > 1:
        h = w // 2
        lo = lo[:, :h] | lo[:, h:w]
        hi = hi[:, :h] | hi[:, h:w]
        w = h

    cin.wait()
    x = blk[...]
    v = jax.lax.broadcasted_iota(jnp.int32, x.shape, 1)
    sh = v & 31
    bit = jnp.where(v < 32, (lo >> sh) & 1, (hi >> sh) & 1)
    banned = (v < 64) & (bit == 1)
    blk[...] = jnp.where(banned, jnp.float32(-jnp.inf), x)
    cout = pltpu.make_async_copy(blk, out_hbm.at[:, :_MASK_W], sem_out)
    cout.start()
    big.wait()
    cout.wait()


@functools.partial(jax.jit, static_argnums=(2,))
def _run(tokens, lprobs, n_rows):
    return pl.pallas_call(
        _ngram_kernel,
        in_specs=[
            pl.BlockSpec(memory_space=pltpu.MemorySpace.VMEM),
            pl.BlockSpec(memory_space=pltpu.MemorySpace.HBM),
        ],
        out_specs=pl.BlockSpec(memory_space=pltpu.MemorySpace.HBM),
        out_shape=jax.ShapeDtypeStruct(lprobs.shape, lprobs.dtype),
        scratch_shapes=[
            pltpu.VMEM((n_rows, _MASK_W), jnp.float32),
            pltpu.SemaphoreType.DMA,
            pltpu.SemaphoreType.DMA,
            pltpu.SemaphoreType.DMA,
        ],
    )(tokens, lprobs)


def kernel(tokens, lprobs, bsz, beam_size, step):
    return _run(tokens, lprobs, lprobs.shape[0])
